# baseline (speedup 1.0000x reference)
import functools

import jax
import jax.numpy as jnp
from jax import lax
from jax.experimental import pallas as pl
from jax.experimental.pallas import tpu as pltpu
from jax.experimental.pallas import tpu_sc as plsc

_PERMUTATION_COUNT = 2


def _make_sc_kernel(B, L, D, O, NC, NS):
    NW = NC * NS
    rpw = O // NW
    groups = rpw // 16

    mesh = plsc.VectorSubcoreMesh(core_axis_name="c", subcore_axis_name="s")

    @functools.partial(
        pl.kernel,
        out_type=jax.ShapeDtypeStruct((O, D), jnp.float32),
        mesh=mesh,
        scratch_types=[
            pltpu.VMEM((L, rpw), jnp.int32),
            pltpu.VMEM((rpw,), jnp.int32),
            pltpu.VMEM((rpw, D), jnp.float32),
            pltpu.SemaphoreType.DMA,
        ],
        compiler_params=pltpu.CompilerParams(use_tc_tiling_on_sc=False),
    )
    def sc_kernel(mask_hbm, emb_hbm, out_hbm, mask_v, idx_v, rows_v, sem):
        wid = lax.axis_index("s") * NC + lax.axis_index("c")
        base = wid * rpw
        pltpu.sync_copy(mask_hbm.at[wid], mask_v)
        lane = lax.iota(jnp.int32, 16)
        for g in range(groups):
            def body(l, acc):
                return acc + mask_v[l, pl.ds(g * 16, 16)]
            acc = lax.fori_loop(0, L, body, jnp.zeros((16,), jnp.int32))
            b = (base + g * 16 + lane) * _PERMUTATION_COUNT
            idx_v[pl.ds(g * 16, 16)] = b * L + acc - 1
        pltpu.async_copy(emb_hbm.at[idx_v], rows_v, sem).wait()
        pltpu.sync_copy(rows_v, out_hbm.at[pl.ds(base, rpw)])

    return sc_kernel


def kernel(embeddings, labels, embeddings_mask, labels_mask):
    B, L, D = embeddings.shape
    O = len(range(0, B - 1, _PERMUTATION_COUNT))
    info = plsc.get_sparse_core_info()
    NC, NS = info.num_cores, info.num_subcores
    NW = NC * NS
    rpw = O // NW

    mask_sel = labels_mask[0:B - 1:_PERMUTATION_COUNT].astype(jnp.int32)
    mask_tiles = mask_sel.reshape(NW, rpw, L).transpose(0, 2, 1)
    emb_flat = embeddings.reshape(B * L, D)

    out = _make_sc_kernel(B, L, D, O, NC, NS)(mask_tiles, emb_flat)
    return out.reshape(O, 1, D)


# speedup vs baseline: 1.0024x; 1.0024x over previous
import functools

import jax
import jax.numpy as jnp
from jax import lax
from jax.experimental import pallas as pl
from jax.experimental.pallas import tpu as pltpu
from jax.experimental.pallas import tpu_sc as plsc

_PERMUTATION_COUNT = 2


def _make_sc_kernel(B, L, D, O, NC, NS):
    NW = NC * NS
    rpw = O // NW
    WPR = L // 4
    nfull = WPR // 16
    rem = WPR % 16
    bpw = rpw * _PERMUTATION_COUNT

    mesh = plsc.VectorSubcoreMesh(core_axis_name="c", subcore_axis_name="s")

    @functools.partial(
        pl.kernel,
        out_type=jax.ShapeDtypeStruct((O, D), jnp.float32),
        mesh=mesh,
        scratch_types=[
            pltpu.VMEM((bpw, WPR), jnp.int32),
            pltpu.VMEM((rpw,), jnp.int32),
            pltpu.VMEM((rpw, D), jnp.float32),
            pltpu.SemaphoreType.DMA,
        ],
        compiler_params=pltpu.CompilerParams(
            use_tc_tiling_on_sc=False, needs_layout_passes=False
        ),
    )
    def sc_kernel(mask_hbm, emb_hbm, out_hbm, mask_v, idx_v, rows_v, sem):
        wid = lax.axis_index("s") * NC + lax.axis_index("c")
        base = wid * rpw
        pltpu.sync_copy(mask_hbm.at[pl.ds(wid * bpw, bpw)], mask_v)

        lane = lax.iota(jnp.int32, 16)
        tail_keep = lane >= (16 - rem)
        for g in range(rpw // 16):
            vec = jnp.zeros((16,), jnp.int32)
            for r in range(16):
                row = g * 16 + r
                src = row * _PERMUTATION_COUNT
                acc = mask_v[src, pl.ds(0, 16)]
                for c in range(1, nfull):
                    acc = acc + mask_v[src, pl.ds(c * 16, 16)]
                if rem:
                    t = mask_v[src, pl.ds(WPR - 16, 16)]
                    acc = acc + jnp.where(tail_keep, t, 0)
                s = jnp.sum(acc)
                cnt = ((s & 0xFF) + ((s >> 8) & 0xFF)
                       + ((s >> 16) & 0xFF) + ((s >> 24) & 0xFF))
                vec = jnp.where(
                    lane == r,
                    (base + row) * _PERMUTATION_COUNT * L + cnt - 1,
                    vec,
                )
            idx_v[pl.ds(g * 16, 16)] = vec
        pltpu.async_copy(emb_hbm.at[idx_v], rows_v, sem).wait()
        pltpu.sync_copy(rows_v, out_hbm.at[pl.ds(base, rpw)])

    return sc_kernel


def kernel(embeddings, labels, embeddings_mask, labels_mask):
    B, L, D = embeddings.shape
    O = len(range(0, B - 1, _PERMUTATION_COUNT))
    info = plsc.get_sparse_core_info()
    NC, NS = info.num_cores, info.num_subcores

    mask_words = lax.bitcast_convert_type(
        labels_mask.reshape(B, L // 4, 4).astype(jnp.uint8), jnp.int32
    )
    emb_flat = embeddings.reshape(B * L, D)

    out = _make_sc_kernel(B, L, D, O, NC, NS)(mask_words, emb_flat)
    return out.reshape(O, 1, D)


# speedup vs baseline: 13.3332x; 13.3010x over previous
import functools

import jax
import jax.numpy as jnp
from jax import lax
from jax.experimental import pallas as pl
from jax.experimental.pallas import tpu as pltpu
from jax.experimental.pallas import tpu_sc as plsc

_PERMUTATION_COUNT = 2


def _make_sc_kernel(B, L, D, O, NC, NS):
    NW = NC * NS
    rpw = O // NW
    LT = L // 8
    s_l = (D // 8) * (B // 128) * 8 * 128
    s_dt = (B // 128) * 8 * 128
    s_bt = 8 * 128

    mesh = plsc.VectorSubcoreMesh(core_axis_name="c", subcore_axis_name="s")

    @functools.partial(
        pl.kernel,
        out_type=jax.ShapeDtypeStruct((O * D,), jnp.float32),
        mesh=mesh,
        scratch_types=[
            pltpu.VMEM((LT, 256), jnp.int32),
            pltpu.VMEM((rpw // 2, 128), jnp.int32),
            pltpu.VMEM((rpw * D,), jnp.float32),
            pltpu.SemaphoreType.DMA,
        ],
        compiler_params=pltpu.CompilerParams(
            use_tc_tiling_on_sc=False, needs_layout_passes=False
        ),
    )
    def sc_kernel(mask_hbm, emb_hbm, out_hbm, mask_v, idx_v, dst_v, sem):
        wid = lax.axis_index("s") * NC + lax.axis_index("c")
        pltpu.sync_copy(mask_hbm.at[:, pl.ds(wid * 256, 256)], mask_v)

        lane = lax.iota(jnp.int32, 16)
        accs = []
        for grp in range(16):
            acc = mask_v[0, pl.ds(grp * 16, 16)]
            for lt in range(1, LT):
                acc = acc + mask_v[lt, pl.ds(grp * 16, 16)]
            accs.append(acc)
        cnts = []
        for g in range(8):
            c = jnp.zeros((16,), jnp.int32)
            for a in (accs[g], accs[8 + g]):
                c = (c + (a & 0xFF) + ((a >> 8) & 0xFF)
                     + ((a >> 16) & 0xFF) + ((a >> 24) & 0xFF))
            cnts.append(c)

        his = []
        for dg in range(D // 16):
            d_vec = dg * 16 + lane
            his.append((d_vec >> 3) * s_dt + (d_vec & 7) * 128)
        w_off = wid * s_bt

        for oo in range(rpw):
            bc = _PERMUTATION_COUNT * oo
            cnt = cnts[bc // 16][bc % 16]
            l_idx = jnp.maximum(cnt, 1) - 1
            row_base = l_idx * s_l + w_off + bc
            for dg in range(D // 16):
                pos = oo * D + dg * 16
                idx_v[pos // 128, pl.ds(pos % 128, 16)] = row_base + his[dg]

        copies = [
            pltpu.async_copy(
                emb_hbm.at[idx_v.at[j]], dst_v.at[pl.ds(j * 128, 128)], sem
            )
            for j in range(rpw * D // 128)
        ]
        for cp in copies:
            cp.wait()
        pltpu.sync_copy(dst_v, out_hbm.at[pl.ds(wid * rpw * D, rpw * D)])

    return sc_kernel


def kernel(embeddings, labels, embeddings_mask, labels_mask):
    B, L, D = embeddings.shape
    O = len(range(0, B - 1, _PERMUTATION_COUNT))
    info = plsc.get_sparse_core_info()
    NC, NS = info.num_cores, info.num_subcores

    emb_phys = (
        embeddings.reshape(B // 128, 128, L, D // 8, 8)
        .transpose(2, 3, 0, 4, 1)
        .reshape(B * L * D)
    )
    mask_words = lax.bitcast_convert_type(
        labels_mask.reshape(B // 128, 128, L // 8, 2, 4)
        .transpose(2, 0, 3, 1, 4)
        .astype(jnp.uint8),
        jnp.int32,
    ).reshape(L // 8, (B // 128) * 2 * 128)

    out = _make_sc_kernel(B, L, D, O, NC, NS)(mask_words, emb_phys)
    return out.reshape(O, 1, D)


# speedup vs baseline: 13.9662x; 1.0475x over previous
import functools

import jax
import jax.numpy as jnp
from jax import lax
from jax.experimental import pallas as pl
from jax.experimental.pallas import tpu as pltpu
from jax.experimental.pallas import tpu_sc as plsc

_PERMUTATION_COUNT = 2


def _make_sc_kernel(B, L, D, O, NC, NS):
    NW = NC * NS
    rpw = O // NW
    WPC = L // 4
    s_l = (D // 8) * (B // 128) * 8 * 128
    s_dt = (B // 128) * 8 * 128
    s_bt = 8 * 128

    mesh = plsc.VectorSubcoreMesh(core_axis_name="c", subcore_axis_name="s")

    @functools.partial(
        pl.kernel,
        out_type=jax.ShapeDtypeStruct((D // 8, O // 16, 128), jnp.float32),
        mesh=mesh,
        scratch_types=[
            pltpu.VMEM((WPC, 128), jnp.int32),
            pltpu.VMEM((D, rpw), jnp.int32),
            pltpu.VMEM((D // 8, 8, rpw), jnp.float32),
            pltpu.SemaphoreType.DMA,
        ],
        compiler_params=pltpu.CompilerParams(
            use_tc_tiling_on_sc=False, needs_layout_passes=False
        ),
    )
    def sc_kernel(mask_hbm, emb_hbm, out_hbm, mask_v, idx_v, dst_v, sem):
        wid = lax.axis_index("s") * NC + lax.axis_index("c")
        pltpu.sync_copy(mask_hbm.at[:, pl.ds(wid * 128, 128)], mask_v)

        lane = lax.iota(jnp.int32, 16)
        cnts = []
        for g in range(8):
            acc = mask_v[0, pl.ds(g * 16, 16)]
            for r in range(1, WPC):
                acc = acc + mask_v[r, pl.ds(g * 16, 16)]
            cnts.append((acc & 0xFF) + ((acc >> 8) & 0xFF)
                        + ((acc >> 16) & 0xFF) + ((acc >> 24) & 0xFF))

        lbs = []
        for g in range(4):
            v = jnp.zeros((16,), jnp.int32)
            for r in range(16):
                bc = _PERMUTATION_COUNT * (g * 16 + r)
                v = jnp.where(lane == r, cnts[bc // 16][bc % 16], v)
            l_idx = jnp.maximum(v, 1) - 1
            lbs.append(l_idx * s_l + wid * s_bt
                       + _PERMUTATION_COUNT * (g * 16) + _PERMUTATION_COUNT * lane)

        for d in range(D):
            hi = (d >> 3) * s_dt + (d & 7) * 128
            for g in range(4):
                idx_v[d, pl.ds(g * 16, 16)] = lbs[g] + hi

        copies = [
            pltpu.async_copy(
                emb_hbm.at[idx_v.at[d]], dst_v.at[d >> 3, d & 7], sem
            )
            for d in range(D)
        ]
        for cp in copies:
            cp.wait()
        pltpu.sync_copy(
            dst_v,
            out_hbm.at[:, pl.ds((wid >> 1) * 8, 8),
                       pl.ds((wid & 1) * rpw, rpw)],
        )

    return sc_kernel


def kernel(embeddings, labels, embeddings_mask, labels_mask):
    B, L, D = embeddings.shape
    O = len(range(0, B - 1, _PERMUTATION_COUNT))
    info = plsc.get_sparse_core_info()
    NC, NS = info.num_cores, info.num_subcores

    emb_phys = (
        embeddings.reshape(B // 128, 128, L, D // 8, 8)
        .transpose(2, 3, 0, 4, 1)
        .reshape(B * L * D)
    )
    mask_words = lax.bitcast_convert_type(
        labels_mask.view(jnp.uint8)
        .reshape(B // 128, 128, L // 8, 2, 4)
        .transpose(2, 3, 0, 1, 4),
        jnp.int32,
    ).reshape(L // 4, B)

    out = _make_sc_kernel(B, L, D, O, NC, NS)(mask_words, emb_phys)
    return (
        out.reshape(D // 8, O // 128, 8, 128)
        .transpose(1, 3, 0, 2)
        .reshape(O, 1, D)
    )


# speedup vs baseline: 15.7735x; 1.1294x over previous
import functools

import jax
import jax.numpy as jnp
from jax import lax
from jax.experimental import pallas as pl
from jax.experimental.pallas import tpu as pltpu
from jax.experimental.pallas import tpu_sc as plsc

_PERMUTATION_COUNT = 2


def _make_sc_kernel(B, L, D, O, NC, NS):
    NW = NC * NS
    rpw = O // NW
    LT = L // 8
    s_l = (D // 8) * (B // 128) * 8 * 128
    s_dt = (B // 128) * 8 * 128
    s_bt = 8 * 128

    mesh = plsc.VectorSubcoreMesh(core_axis_name="c", subcore_axis_name="s")

    @functools.partial(
        pl.kernel,
        out_type=jax.ShapeDtypeStruct((D // 8, O // 16, 128), jnp.float32),
        mesh=mesh,
        scratch_types=[
            pltpu.VMEM((LT, 8 * 128), jnp.int32),
            pltpu.VMEM((D, rpw), jnp.int32),
            pltpu.VMEM((D // 8, 8, rpw), jnp.float32),
            pltpu.SemaphoreType.DMA,
        ],
        compiler_params=pltpu.CompilerParams(
            use_tc_tiling_on_sc=False, needs_layout_passes=False
        ),
    )
    def sc_kernel(mask_hbm, emb_hbm, out_hbm, mask_v, idx_v, dst_v, sem):
        wid = lax.axis_index("s") * NC + lax.axis_index("c")
        pltpu.sync_copy(mask_hbm.at[:, pl.ds(wid * 1024, 1024)], mask_v)

        lane = lax.iota(jnp.int32, 16)

        def sum_body(r, accs):
            out = []
            for g in range(8):
                a = accs[g]
                for c in range(8):
                    a = a + mask_v[r, pl.ds(c * 128 + g * 16, 16)]
                out.append(a)
            return tuple(out)

        cnts = lax.fori_loop(
            0, LT, sum_body, tuple(jnp.zeros((16,), jnp.int32) for _ in range(8))
        )

        lbs = []
        for g in range(4):
            v = jnp.zeros((16,), jnp.int32)
            for r in range(16):
                bc = _PERMUTATION_COUNT * (g * 16 + r)
                v = jnp.where(lane == r, cnts[bc // 16][bc % 16], v)
            l_idx = jnp.maximum(v, 1) - 1
            lbs.append(l_idx * s_l + wid * s_bt
                       + _PERMUTATION_COUNT * (g * 16) + _PERMUTATION_COUNT * lane)

        for d in range(D):
            hi = (d >> 3) * s_dt + (d & 7) * 128
            for g in range(4):
                idx_v[d, pl.ds(g * 16, 16)] = lbs[g] + hi

        copies = [
            pltpu.async_copy(
                emb_hbm.at[idx_v.at[d]], dst_v.at[d >> 3, d & 7], sem
            )
            for d in range(D)
        ]
        for cp in copies:
            cp.wait()
        pltpu.sync_copy(
            dst_v,
            out_hbm.at[:, pl.ds((wid >> 1) * 8, 8),
                       pl.ds((wid & 1) * rpw, rpw)],
        )

    return sc_kernel


def kernel(embeddings, labels, embeddings_mask, labels_mask):
    B, L, D = embeddings.shape
    O = len(range(0, B - 1, _PERMUTATION_COUNT))
    info = plsc.get_sparse_core_info()
    NC, NS = info.num_cores, info.num_subcores

    emb_phys = (
        embeddings.reshape(B // 128, 128, L, D // 8, 8)
        .transpose(2, 3, 0, 4, 1)
        .reshape(B * L * D)
    )
    mask_i32 = (
        labels_mask.astype(jnp.int32)
        .reshape(B // 128, 128, L // 8, 8)
        .transpose(2, 0, 3, 1)
        .reshape(L // 8, (B // 128) * 8 * 128)
    )

    out = _make_sc_kernel(B, L, D, O, NC, NS)(mask_i32, emb_phys)
    return (
        out.reshape(D // 8, O // 128, 8, 128)
        .transpose(1, 3, 0, 2)
        .reshape(O, 1, D)
    )


# speedup vs baseline: 16.6876x; 1.0579x over previous
import functools

import jax
import jax.numpy as jnp
from jax import lax
from jax.experimental import pallas as pl
from jax.experimental.pallas import tpu as pltpu
from jax.experimental.pallas import tpu_sc as plsc

_PERMUTATION_COUNT = 2


def _tc_lengths(L, B):
    def body(m_ref, o_ref):
        cnt = jnp.sum(m_ref[...].astype(jnp.int32), axis=0, keepdims=True)
        o_ref[...] = jnp.broadcast_to(cnt, (8, B))

    return pl.pallas_call(
        body,
        out_shape=jax.ShapeDtypeStruct((8, B), jnp.int32),
    )


def _make_sc_kernel(B, L, D, O, NC, NS):
    NW = NC * NS
    rpw = O // NW
    s_l = (D // 8) * (B // 128) * 8 * 128
    s_dt = (B // 128) * 8 * 128
    s_bt = 8 * 128

    mesh = plsc.VectorSubcoreMesh(core_axis_name="c", subcore_axis_name="s")

    @functools.partial(
        pl.kernel,
        out_type=jax.ShapeDtypeStruct((D // 8, O // 16, 128), jnp.float32),
        mesh=mesh,
        scratch_types=[
            pltpu.VMEM((128,), jnp.int32),
            pltpu.VMEM((D, rpw), jnp.int32),
            pltpu.VMEM((D // 8, 8, rpw), jnp.float32),
            pltpu.SemaphoreType.DMA,
        ],
        compiler_params=pltpu.CompilerParams(
            use_tc_tiling_on_sc=False, needs_layout_passes=False
        ),
    )
    def sc_kernel(cnt_hbm, emb_hbm, out_hbm, cnt_v, idx_v, dst_v, sem):
        wid = lax.axis_index("s") * NC + lax.axis_index("c")
        pltpu.sync_copy(cnt_hbm.at[pl.ds(wid * 1024, 128)], cnt_v)

        lane = lax.iota(jnp.int32, 16)
        cnts = [cnt_v[pl.ds(k * 16, 16)] for k in range(8)]

        lbs = []
        for g in range(4):
            v = jnp.zeros((16,), jnp.int32)
            for r in range(16):
                bc = _PERMUTATION_COUNT * (g * 16 + r)
                v = jnp.where(lane == r, cnts[bc // 16][bc % 16], v)
            l_idx = jnp.maximum(v, 1) - 1
            lbs.append(l_idx * s_l + wid * s_bt
                       + _PERMUTATION_COUNT * (g * 16) + _PERMUTATION_COUNT * lane)

        for d in range(D):
            hi = (d >> 3) * s_dt + (d & 7) * 128
            for g in range(4):
                idx_v[d, pl.ds(g * 16, 16)] = lbs[g] + hi

        copies = [
            pltpu.async_copy(
                emb_hbm.at[idx_v.at[d]], dst_v.at[d >> 3, d & 7], sem
            )
            for d in range(D)
        ]
        for cp in copies:
            cp.wait()
        pltpu.sync_copy(
            dst_v,
            out_hbm.at[:, pl.ds((wid >> 1) * 8, 8),
                       pl.ds((wid & 1) * rpw, rpw)],
        )

    return sc_kernel


def kernel(embeddings, labels, embeddings_mask, labels_mask):
    B, L, D = embeddings.shape
    O = len(range(0, B - 1, _PERMUTATION_COUNT))
    info = plsc.get_sparse_core_info()
    NC, NS = info.num_cores, info.num_subcores

    emb_phys = (
        embeddings.reshape(B // 128, 128, L, D // 8, 8)
        .transpose(2, 3, 0, 4, 1)
        .reshape(B * L * D)
    )
    cnt = (
        _tc_lengths(L, B)(labels_mask.T.view(jnp.int8))
        .reshape(8, B // 128, 128)
        .transpose(1, 0, 2)
        .reshape(8 * B)
    )

    out = _make_sc_kernel(B, L, D, O, NC, NS)(cnt, emb_phys)
    return (
        out.reshape(D // 8, O // 128, 8, 128)
        .transpose(1, 3, 0, 2)
        .reshape(O, 1, D)
    )


# speedup vs baseline: 17.0954x; 1.0244x over previous
import functools

import jax
import jax.numpy as jnp
from jax import lax
from jax.experimental import pallas as pl
from jax.experimental.pallas import tpu as pltpu
from jax.experimental.pallas import tpu_sc as plsc

_PERMUTATION_COUNT = 2


def _tc_lengths(L, B):
    def body(m_ref, o_ref):
        cnt = jnp.sum(m_ref[...].astype(jnp.int32), axis=0, keepdims=True)
        o_ref[...] = jnp.broadcast_to(cnt, (8, B))

    return pl.pallas_call(
        body,
        out_shape=jax.ShapeDtypeStruct((8, B), jnp.int32),
    )


def _make_sc_kernel(B, L, D, O, NC, NS):
    NW = NC * NS
    rpw = O // NW
    s_l = (D // 8) * (B // 128) * 8 * 128
    s_dt = (B // 128) * 8 * 128
    s_bt = 8 * 128

    mesh = plsc.VectorSubcoreMesh(core_axis_name="c", subcore_axis_name="s")

    @functools.partial(
        pl.kernel,
        out_type=jax.ShapeDtypeStruct((D // 8, O // 16, 128), jnp.float32),
        mesh=mesh,
        scratch_types=[
            pltpu.VMEM((128,), jnp.int32),
            pltpu.VMEM((D, rpw), jnp.int32),
            pltpu.VMEM((D // 8, 8, rpw), jnp.float32),
            pltpu.SemaphoreType.DMA,
        ],
        compiler_params=pltpu.CompilerParams(
            use_tc_tiling_on_sc=False, needs_layout_passes=False
        ),
    )
    def sc_kernel(cnt_hbm, emb_hbm, out_hbm, cnt_v, idx_v, dst_v, sem):
        wid = lax.axis_index("s") * NC + lax.axis_index("c")
        pltpu.sync_copy(cnt_hbm.at[pl.ds(wid * 1024, 128)], cnt_v)

        lane = lax.iota(jnp.int32, 16)
        cnts = [cnt_v[pl.ds(k * 16, 16)] for k in range(8)]

        lbs = []
        for g in range(4):
            v = jnp.zeros((16,), jnp.int32)
            for r in range(16):
                bc = _PERMUTATION_COUNT * (g * 16 + r)
                v = jnp.where(lane == r, cnts[bc // 16][bc % 16], v)
            l_idx = jnp.maximum(v, 1) - 1
            lbs.append(l_idx * s_l + wid * s_bt
                       + _PERMUTATION_COUNT * (g * 16) + _PERMUTATION_COUNT * lane)

        def idx_body(d, _):
            hi = (d >> 3) * s_dt + (d & 7) * 128
            for g in range(4):
                idx_v[d, pl.ds(g * 16, 16)] = lbs[g] + hi
            return 0

        lax.fori_loop(0, D, idx_body, 0)

        def fire_body(d, _):
            pltpu.async_copy(
                emb_hbm.at[idx_v.at[d]], dst_v.at[d >> 3, d & 7], sem
            )
            return 0

        def drain_body(d, _):
            pltpu.make_async_copy(
                emb_hbm.at[idx_v.at[d]], dst_v.at[d >> 3, d & 7], sem
            ).wait()
            return 0

        lax.fori_loop(0, D, fire_body, 0)
        lax.fori_loop(0, D, drain_body, 0)
        pltpu.sync_copy(
            dst_v,
            out_hbm.at[:, pl.ds((wid >> 1) * 8, 8),
                       pl.ds((wid & 1) * rpw, rpw)],
        )

    return sc_kernel


def kernel(embeddings, labels, embeddings_mask, labels_mask):
    B, L, D = embeddings.shape
    O = len(range(0, B - 1, _PERMUTATION_COUNT))
    info = plsc.get_sparse_core_info()
    NC, NS = info.num_cores, info.num_subcores

    emb_phys = (
        embeddings.reshape(B // 128, 128, L, D // 8, 8)
        .transpose(2, 3, 0, 4, 1)
        .reshape(B * L * D)
    )
    cnt = (
        _tc_lengths(L, B)(labels_mask.T.view(jnp.int8))
        .reshape(8, B // 128, 128)
        .transpose(1, 0, 2)
        .reshape(8 * B)
    )

    out = _make_sc_kernel(B, L, D, O, NC, NS)(cnt, emb_phys)
    return (
        out.reshape(D // 8, O // 128, 8, 128)
        .transpose(1, 3, 0, 2)
        .reshape(O, 1, D)
    )


# speedup vs baseline: 17.9877x; 1.0522x over previous
import functools

import jax
import jax.numpy as jnp
from jax import lax
from jax.experimental import pallas as pl
from jax.experimental.pallas import tpu as pltpu
from jax.experimental.pallas import tpu_sc as plsc

_PERMUTATION_COUNT = 2


def _tc_lengths(L, B):
    def body(m_ref, o_ref):
        cnt = jnp.sum(m_ref[...].astype(jnp.int32), axis=0, keepdims=True)
        o_ref[...] = jnp.broadcast_to(cnt, (8, B))

    return pl.pallas_call(
        body,
        out_shape=jax.ShapeDtypeStruct((8, B), jnp.int32),
    )


def _make_sc_kernel(B, L, D, O, NC, NS):
    NW = NC * NS
    rpw = O // NW
    s_l = (D // 8) * (B // 128) * 8 * 128
    s_dt = (B // 128) * 8 * 128
    s_bt = 8 * 128

    mesh = plsc.VectorSubcoreMesh(core_axis_name="c", subcore_axis_name="s")

    gdnums = lax.GatherDimensionNumbers(
        offset_dims=(), collapsed_slice_dims=(0,), start_index_map=(0,)
    )

    def _lane_gather(v, idx16):
        return lax.gather(
            v, idx16[:, None], gdnums, slice_sizes=(1,),
            mode=lax.GatherScatterMode.PROMISE_IN_BOUNDS,
        )

    @functools.partial(
        pl.kernel,
        out_type=jax.ShapeDtypeStruct((D // 8, O // 16, 128), jnp.float32),
        mesh=mesh,
        scratch_types=[
            pltpu.VMEM((128,), jnp.int32),
            pltpu.VMEM((D, rpw), jnp.int32),
            pltpu.VMEM((D // 8, 8 * 128), jnp.float32),
            pltpu.VMEM((D // 8, 8, rpw), jnp.float32),
            pltpu.SemaphoreType.DMA,
        ],
        compiler_params=pltpu.CompilerParams(
            use_tc_tiling_on_sc=False, needs_layout_passes=False
        ),
    )
    def sc_kernel(cnt_hbm, emb_hbm, out_hbm,
                  cnt_v, idx_v, blk_v, dst_v, sem):
        wid = lax.axis_index("s") * NC + lax.axis_index("c")
        pltpu.sync_copy(cnt_hbm.at[pl.ds(wid * 1024, 128)], cnt_v)

        lane = lax.iota(jnp.int32, 16)
        cnts = [cnt_v[pl.ds(k * 16, 16)] for k in range(8)]

        lbs, lvs = [], []
        for g in range(4):
            v = jnp.zeros((16,), jnp.int32)
            for r in range(16):
                bc = _PERMUTATION_COUNT * (g * 16 + r)
                v = jnp.where(lane == r, cnts[bc // 16][bc % 16], v)
            l_idx = jnp.maximum(v, 1) - 1
            lvs.append(l_idx)
            lbs.append(l_idx * s_l + wid * s_bt
                       + _PERMUTATION_COUNT * (g * 16) + _PERMUTATION_COUNT * lane)

        lmin = jnp.min(jnp.minimum(jnp.minimum(lvs[0], lvs[1]),
                                   jnp.minimum(lvs[2], lvs[3])))
        lmax = jnp.max(jnp.maximum(jnp.maximum(lvs[0], lvs[1]),
                                   jnp.maximum(lvs[2], lvs[3])))
        uniform = lmin == lmax

        @pl.when(uniform)
        def _fast():
            base = lmin * s_l + wid * s_bt
            for dt in range(D // 8):
                pltpu.sync_copy(
                    emb_hbm.at[pl.ds(base + dt * s_dt, 8 * 128)],
                    blk_v.at[dt],
                )
            ev = jnp.where(lane < 8, lane * 2, lane * 2 - 16)

            def dt_body(dt, _):
                for dr in range(8):
                    vs = [blk_v[dt, pl.ds(dr * 128 + k * 16, 16)]
                          for k in range(8)]
                    for k in range(4):
                        g1 = _lane_gather(vs[2 * k], ev)
                        g2 = _lane_gather(vs[2 * k + 1], ev)
                        dst_v[dt, dr, pl.ds(k * 16, 16)] = (
                            jnp.where(lane < 8, g1, g2))
                return 0

            lax.fori_loop(0, D // 8, dt_body, 0)

        @pl.when(jnp.logical_not(uniform))
        def _slow():
            def idx_body(d, _):
                hi = (d >> 3) * s_dt + (d & 7) * 128
                for g in range(4):
                    idx_v[d, pl.ds(g * 16, 16)] = lbs[g] + hi
                return 0

            lax.fori_loop(0, D, idx_body, 0)

            def fire_body(d, _):
                pltpu.async_copy(
                    emb_hbm.at[idx_v.at[d]], dst_v.at[d >> 3, d & 7], sem
                )
                return 0

            def drain_body(d, _):
                pltpu.make_async_copy(
                    emb_hbm.at[idx_v.at[d]], dst_v.at[d >> 3, d & 7], sem
                ).wait()
                return 0

            lax.fori_loop(0, D, fire_body, 0)
            lax.fori_loop(0, D, drain_body, 0)
        pltpu.sync_copy(
            dst_v,
            out_hbm.at[:, pl.ds((wid >> 1) * 8, 8),
                       pl.ds((wid & 1) * rpw, rpw)],
        )

    return sc_kernel


def kernel(embeddings, labels, embeddings_mask, labels_mask):
    B, L, D = embeddings.shape
    O = len(range(0, B - 1, _PERMUTATION_COUNT))
    info = plsc.get_sparse_core_info()
    NC, NS = info.num_cores, info.num_subcores

    emb_phys = (
        embeddings.reshape(B // 128, 128, L, D // 8, 8)
        .transpose(2, 3, 0, 4, 1)
        .reshape(B * L * D)
    )
    cnt = (
        _tc_lengths(L, B)(labels_mask.T.view(jnp.int8))
        .reshape(8, B // 128, 128)
        .transpose(1, 0, 2)
        .reshape(8 * B)
    )

    out = _make_sc_kernel(B, L, D, O, NC, NS)(cnt, emb_phys)
    return (
        out.reshape(D // 8, O // 128, 8, 128)
        .transpose(1, 3, 0, 2)
        .reshape(O, 1, D)
    )
